# baseline (speedup 1.0000x reference)
import functools

import jax
import jax.numpy as jnp
from jax import lax
from jax.experimental import pallas as pl
from jax.experimental.pallas import tpu as pltpu
from jax.experimental.pallas import tpu_sc as plsc

_N = 10000
_E = 320000
_D = 128
_ED = 16
_H = 128
_B = 512
_CD = 64
_NCELL = 1000
_CE = 32
_MH = 256

_SC_CORES = 2
_SC_TILES = 16
_HD = _D // 2
_EPT = _E // _SC_TILES
_CHUNK = 80
_NCH = _EPT // _CHUNK
_STRIDE = 624
_SWIDTH = 640
_ZROWS = 128

_F32 = jnp.float32
_BF16 = jnp.bfloat16

_PERM = tuple(32 * (j // 32) + (j % 32) // 2 + 16 * (j % 2)
              for j in range(64))
_PERM_FULL = _PERM + tuple(64 + p for p in _PERM)


def _sc_aggr_body(x_hbm, src_hbm, dst_hbm, ew_hbm, out_hbm,
                  src_v, dst_v, x_v, ew_v, m_v, zero_v, gsem, esem, ssem,
                  aggr_sh):
    cid = lax.axis_index("c")
    sid = lax.axis_index("s")

    pltpu.sync_copy(src_hbm.at[sid], src_v)
    pltpu.sync_copy(dst_hbm.at[sid], dst_v)

    def _zrow(r, c):
        for cc in range(_HD // 16):
            zero_v[r, pl.ds(cc * 16, 16)] = jnp.zeros((16,), _F32)
        return c
    lax.fori_loop(0, _ZROWS, _zrow, 0)
    zbase = pl.multiple_of(sid * _STRIDE, 8)
    for j in range(_SWIDTH // _ZROWS):
        pltpu.sync_copy(zero_v, aggr_sh.at[pl.ds(zbase + j * _ZROWS, _ZROWS)])
    plsc.subcore_barrier()

    def _fire(ch, b):
        pltpu.async_copy(x_hbm.at[cid].at[src_v.at[ch]], x_v.at[b], gsem.at[b])
        ebase = sid * (_EPT // 2) + ch * (_CHUNK // 2)
        pltpu.async_copy(ew_hbm.at[cid, pl.ds(ebase, _CHUNK // 2)],
                         ew_v.at[b], esem.at[b])

    def _wait(b):
        pltpu.make_async_copy(x_hbm.at[cid, pl.ds(0, _CHUNK)], x_v.at[b],
                              gsem.at[b]).wait()
        pltpu.make_async_copy(ew_hbm.at[cid, pl.ds(0, _CHUNK // 2)],
                              ew_v.at[b], esem.at[b]).wait()

    def _swait(ch, b):
        pltpu.make_async_copy(m_v.at[b], aggr_sh.at[dst_v.at[ch]],
                              ssem.at[b]).wait()

    def _process(i, ch, b):
        _wait(b)

        @pl.when(i > 0)
        def _():
            _swait(ch, b)

        def _rows(rq, c2):
            for rr in range(5):
                r = rq * 5 + rr
                for h in range(2):
                    e = r * 2 + h
                    for cc in range(_HD // 16):
                        s = pl.ds(cc * 16, 16)
                        se = pl.ds(h * _HD + cc * 16, 16)
                        m_v[b, e, s] = jnp.maximum(x_v[b, e, s] +
                                                   ew_v[b, r, se], 0.0)
            return c2
        lax.fori_loop(0, _CHUNK // 10, _rows, 0)
        pltpu.async_copy(m_v.at[b], aggr_sh.at[dst_v.at[ch]], ssem.at[b],
                         add=True)

    _fire(0, 0)
    npair = _NCH // 2

    def _pair(i, c):
        c0 = 2 * i
        _fire(c0 + 1, 1)
        _process(i, c0, 0)

        @pl.when(i < npair - 1)
        def _():
            _fire(c0 + 2, 0)
        _process(i, c0 + 1, 1)
        return c
    lax.fori_loop(0, npair, _pair, 0)
    for b in range(2):
        _swait(0, b)
    plsc.subcore_barrier()

    obase = pl.multiple_of(sid * _STRIDE, 8)
    pltpu.sync_copy(aggr_sh.at[pl.ds(obase, _SWIDTH)],
                    out_hbm.at[cid, pl.ds(obase, _SWIDTH)])


def _sc_aggregate(xs, src3, dst3, ews):
    mesh = plsc.VectorSubcoreMesh(core_axis_name="c", subcore_axis_name="s")
    fn = functools.partial(
        pl.kernel,
        out_type=jax.ShapeDtypeStruct((_SC_CORES, _N, _HD), _F32),
        mesh=mesh,
        scratch_types=[
            pltpu.VMEM((_NCH, _CHUNK), jnp.int32),
            pltpu.VMEM((_NCH, _CHUNK), jnp.int32),
            pltpu.VMEM((2, _CHUNK, _HD), _F32),
            pltpu.VMEM((2, _CHUNK // 2, _D), _F32),
            pltpu.VMEM((2, _CHUNK, _HD), _F32),
            pltpu.VMEM((_ZROWS, _HD), _F32),
            pltpu.SemaphoreType.DMA((2,)),
            pltpu.SemaphoreType.DMA((2,)),
            pltpu.SemaphoreType.DMA((2,)),
            pltpu.VMEM_SHARED((_N, _HD), _F32),
        ],
        compiler_params=pltpu.CompilerParams(use_tc_tiling_on_sc=False),
    )(_sc_aggr_body)
    return fn(xs, src3, dst3, ews)


_ET_BLK = 8000


_EP_BLK = _ET_BLK // 2


def _edge_tf_body(eaa_ref, eab_ref, w00_ref, w01_ref, w10_ref, w11_ref,
                  b00_ref, b01_ref, b10_ref, b11_ref, e0_ref, e1_ref):
    eaa = eaa_ref[...]
    eab = eab_ref[...]

    def _pk(w_ref, b_ref):
        w = w_ref[...]
        b = b_ref[...]
        efa = jnp.dot(eaa, w, preferred_element_type=_F32) + b
        efb = jnp.dot(eab, w, preferred_element_type=_F32) + b
        return jnp.concatenate([efa, efb], axis=1)

    e0_ref[...] = jnp.stack([_pk(w00_ref, b00_ref), _pk(w01_ref, b01_ref)])
    e1_ref[...] = jnp.stack([_pk(w10_ref, b10_ref), _pk(w11_ref, b11_ref)])


_EP_GRID = (_E // 2) // _EP_BLK


def _edge_transform(ea, wps, bps):
    return pl.pallas_call(
        _edge_tf_body,
        grid=(_EP_GRID,),
        in_specs=[
            pl.BlockSpec((_EP_BLK, _ED), lambda i: (i, 0)),
            pl.BlockSpec((_EP_BLK, _ED), lambda i: (i + _EP_GRID, 0)),
            pl.BlockSpec((_ED, _HD), lambda i: (0, 0)),
            pl.BlockSpec((_ED, _HD), lambda i: (0, 0)),
            pl.BlockSpec((_ED, _HD), lambda i: (0, 0)),
            pl.BlockSpec((_ED, _HD), lambda i: (0, 0)),
            pl.BlockSpec((1, _HD), lambda i: (0, 0)),
            pl.BlockSpec((1, _HD), lambda i: (0, 0)),
            pl.BlockSpec((1, _HD), lambda i: (0, 0)),
            pl.BlockSpec((1, _HD), lambda i: (0, 0)),
        ],
        out_specs=[
            pl.BlockSpec((2, _EP_BLK, _D), lambda i: (0, i, 0)),
            pl.BlockSpec((2, _EP_BLK, _D), lambda i: (0, i, 0)),
        ],
        out_shape=[
            jax.ShapeDtypeStruct((2, _E // 2, _D), _F32),
            jax.ShapeDtypeStruct((2, _E // 2, _D), _F32),
        ],
    )(ea, ea, *wps, *bps)


_NU_BLK = 2000


def _node_update_body(x_ref, p_ref, w1_ref, b1_ref, w2_ref, b2_ref,
                      u_ref, st_ref):
    i = pl.program_id(0)
    p = p_ref[...]
    h = x_ref[...] + jnp.concatenate([p[0], p[1]], axis=1)
    t = jnp.maximum(jnp.dot(h, w1_ref[...], preferred_element_type=_F32)
                    + b1_ref[...], 0.0)
    u = jnp.dot(t, w2_ref[...], preferred_element_type=_F32) + b2_ref[...]
    u_ref[...] = u
    s = jnp.concatenate([jnp.sum(u, axis=0, keepdims=True),
                         jnp.sum(u * u, axis=0, keepdims=True)], axis=0)

    @pl.when(i == 0)
    def _():
        st_ref[...] = s

    @pl.when(i > 0)
    def _():
        st_ref[...] = st_ref[...] + s


def _node_update(x, parts, w1t, b1, w2t, b2):
    g = _N // _NU_BLK
    return pl.pallas_call(
        _node_update_body,
        grid=(g,),
        in_specs=[
            pl.BlockSpec((_NU_BLK, _D), lambda i: (i, 0)),
            pl.BlockSpec((2, _NU_BLK, _HD), lambda i: (0, i, 0)),
            pl.BlockSpec((_D, _H), lambda i: (0, 0)),
            pl.BlockSpec((1, _H), lambda i: (0, 0)),
            pl.BlockSpec((_H, _H), lambda i: (0, 0)),
            pl.BlockSpec((1, _H), lambda i: (0, 0)),
        ],
        out_specs=[
            pl.BlockSpec((_NU_BLK, _H), lambda i: (i, 0)),
            pl.BlockSpec((2, _H), lambda i: (0, 0)),
        ],
        out_shape=[
            jax.ShapeDtypeStruct((_N, _H), _F32),
            jax.ShapeDtypeStruct((2, _H), _F32),
        ],
    )(x, parts, w1t, b1, w2t, b2)


def _bn_relu_body(u_ref, st_ref, g_ref, b_ref, y_ref, ys_ref):
    st = st_ref[...]
    mu = st[0:1] / _N
    var = st[1:2] / _N - mu * mu
    inv = lax.rsqrt(var + 1e-5) * g_ref[...]
    y = jnp.maximum((u_ref[...] - mu) * inv + b_ref[...], 0.0)
    y_ref[...] = y
    ys_ref[...] = jnp.stack([y[:, :_HD], y[:, _HD:]])


def _bn_relu(u, st, g, b):
    gr = _N // _NU_BLK
    return pl.pallas_call(
        _bn_relu_body,
        grid=(gr,),
        in_specs=[
            pl.BlockSpec((_NU_BLK, _H), lambda i: (i, 0)),
            pl.BlockSpec((2, _H), lambda i: (0, 0)),
            pl.BlockSpec((1, _H), lambda i: (0, 0)),
            pl.BlockSpec((1, _H), lambda i: (0, 0)),
        ],
        out_specs=[
            pl.BlockSpec((_NU_BLK, _H), lambda i: (i, 0)),
            pl.BlockSpec((2, _NU_BLK, _HD), lambda i: (0, i, 0)),
        ],
        out_shape=[
            jax.ShapeDtypeStruct((_N, _H), _F32),
            jax.ShapeDtypeStruct((2, _N, _HD), _F32),
        ],
    )(u, st, g, b)


_PB = _NU_BLK
_PG = _N // _PB


def _pool_body(u_ref, st_ref, g_ref, b_ref, batch_ref, emb_ref,
               acc_ref, cnt_ref):
    i = pl.program_id(0)

    @pl.when(i == 0)
    def _():
        acc_ref[...] = jnp.zeros_like(acc_ref)
        cnt_ref[...] = jnp.zeros_like(cnt_ref)

    st = st_ref[...]
    mu = st[0:1] / _N
    var = st[1:2] / _N - mu * mu
    inv = lax.rsqrt(var + 1e-5) * g_ref[...]
    y = jnp.maximum((u_ref[...] - mu) * inv + b_ref[...], 0.0)
    bb = batch_ref[0]
    oht = (bb == lax.broadcasted_iota(jnp.int32, (_B, _PB), 0)
           ).astype(_F32)
    acc_ref[...] += jnp.dot(oht, y, preferred_element_type=_F32)
    cnt_ref[...] += jnp.sum(oht, axis=1, keepdims=True)

    @pl.when(i == _PG - 1)
    def _():
        emb_ref[...] = acc_ref[...] / jnp.maximum(cnt_ref[...], 1.0)


def _pool(u, st, g, b, batch3):
    return pl.pallas_call(
        _pool_body,
        grid=(_PG,),
        in_specs=[
            pl.BlockSpec((_PB, _H), lambda i: (i, 0)),
            pl.BlockSpec((2, _H), lambda i: (0, 0)),
            pl.BlockSpec((1, _H), lambda i: (0, 0)),
            pl.BlockSpec((1, _H), lambda i: (0, 0)),
            pl.BlockSpec((1, 1, _PB), lambda i: (i, 0, 0)),
        ],
        out_specs=pl.BlockSpec((_B, _H), lambda i: (0, 0)),
        out_shape=jax.ShapeDtypeStruct((_B, _H), _F32),
        scratch_shapes=[
            pltpu.VMEM((_B, _H), _F32),
            pltpu.VMEM((_B, 1), _F32),
        ],
    )(u, st, g, b, batch3)


def _head_body(e1_ref, e2_ref, cv_ref, ci_ref, tab_ref,
               w1_ref, b1_ref, w2_ref, b2_ref, w3_ref, b3_ref, o_ref):
    ci = ci_ref[...]
    oh = (ci == lax.broadcasted_iota(jnp.int32, (_B, _NCELL), 1)
          ).astype(_F32)
    ce = jnp.dot(oh, tab_ref[...], preferred_element_type=_F32)
    comb = jnp.concatenate([e1_ref[...], e2_ref[...], cv_ref[...], ce],
                           axis=1)
    h = jnp.maximum(jnp.dot(comb, w1_ref[...], preferred_element_type=_F32)
                    + b1_ref[...], 0.0)
    h = jnp.maximum(jnp.dot(h, w2_ref[...], preferred_element_type=_F32)
                    + b2_ref[...], 0.0)
    o_ref[...] = jnp.dot(h, w3_ref[...], preferred_element_type=_F32) + b3_ref[...]


def _head(e1, e2, cv, ci_col, tab, w1t, b1, w2t, b2, w3t, b3):
    return pl.pallas_call(
        _head_body,
        out_shape=jax.ShapeDtypeStruct((_B, 1), _F32),
    )(e1, e2, cv, ci_col, tab, w1t, b1, w2t, b2, w3t, b3)


def _encode(x, xs, src3, dst3, ea, batch3, lp):
    ew0, ew1 = _edge_transform(ea, lp["wps"], lp["bps"])
    p0 = _sc_aggregate(xs, src3, dst3, ew0)
    u0, st0 = _node_update(x, p0, lp["w01t"], lp["b01"], lp["w02t"], lp["b02"])
    y0, y0s = _bn_relu(u0, st0, lp["g0"], lp["bb0"])
    p1 = _sc_aggregate(y0s, src3, dst3, ew1)
    u1, st1 = _node_update(y0, p1, lp["w11t"], lp["b11"], lp["w12t"], lp["b12"])
    return _pool(u1, st1, lp["g1"], lp["bb1"], batch3)


def kernel(x1, edge_index1, edge_attr1, batch1, x2, edge_index2, edge_attr2,
           batch2, cancer_vec, cell_idx,
           conv0_We, conv0_be, conv0_W1, conv0_b1, conv0_W2, conv0_b2,
           bn0_g, bn0_b,
           conv1_We, conv1_be, conv1_W1, conv1_b1, conv1_W2, conv1_b2,
           bn1_g, bn1_b,
           cell_table, fc1_W, fc1_b, fc2_W, fc2_b, fc3_W, fc3_b):
    def _pw(wt, c):
        return wt[:, c * _HD:(c + 1) * _HD]

    def _pb(b, c):
        return b.reshape(1, -1)[:, c * _HD:(c + 1) * _HD]

    w0et = conv0_We.T
    w1et = conv1_We.T
    lp = {
        "wps": [_pw(w0et, 0), _pw(w0et, 1), _pw(w1et, 0), _pw(w1et, 1)],
        "bps": [_pb(conv0_be, 0), _pb(conv0_be, 1),
                _pb(conv1_be, 0), _pb(conv1_be, 1)],
        "w01t": conv0_W1.T, "b01": conv0_b1.reshape(1, _H),
        "w02t": conv0_W2.T, "b02": conv0_b2.reshape(1, _H),
        "w11t": conv1_W1.T, "b11": conv1_b1.reshape(1, _H),
        "w12t": conv1_W2.T, "b12": conv1_b2.reshape(1, _H),
        "g0": bn0_g.reshape(1, _H), "bb0": bn0_b.reshape(1, _H),
        "g1": bn1_g.reshape(1, _H), "bb1": bn1_b.reshape(1, _H),
    }
    def _ilv(idx):
        a = idx[:_E // 2].reshape(_SC_TILES, _NCH, _CHUNK // 2)
        bb = idx[_E // 2:].reshape(_SC_TILES, _NCH, _CHUNK // 2)
        return jnp.stack([a, bb], axis=-1).reshape(_SC_TILES, _NCH, _CHUNK)

    src3a = _ilv(edge_index1[0])
    dst3a = _ilv(edge_index1[1])
    src3b = _ilv(edge_index2[0])
    dst3b = _ilv(edge_index2[1])
    batch3a = batch1.reshape(_PG, 1, _PB)
    batch3b = batch2.reshape(_PG, 1, _PB)
    x1s = jnp.stack([x1[:, :_HD], x1[:, _HD:]])
    x2s = jnp.stack([x2[:, :_HD], x2[:, _HD:]])
    emb1 = _encode(x1, x1s, src3a, dst3a, edge_attr1, batch3a, lp)
    emb2 = _encode(x2, x2s, src3b, dst3b, edge_attr2, batch3b, lp)

    return _head(emb1, emb2, cancer_vec, cell_idx.reshape(_B, 1), cell_table,
                 fc1_W.T, fc1_b.reshape(1, _MH),
                 fc2_W.T, fc2_b.reshape(1, _MH // 2),
                 fc3_W.T, fc3_b.reshape(1, 1))


# speedup vs baseline: 1.0628x; 1.0628x over previous
import functools

import jax
import jax.numpy as jnp
from jax import lax
from jax.experimental import pallas as pl
from jax.experimental.pallas import tpu as pltpu
from jax.experimental.pallas import tpu_sc as plsc

_N = 10000
_E = 320000
_D = 128
_ED = 16
_H = 128
_B = 512
_CD = 64
_NCELL = 1000
_CE = 32
_MH = 256

_SC_CORES = 2
_SC_TILES = 16
_HD = _D // 2
_EPT = _E // _SC_TILES
_CHUNK = 80
_NCH = _EPT // _CHUNK
_STRIDE = 624
_SWIDTH = 640
_ZROWS = 128

_F32 = jnp.float32
_BF16 = jnp.bfloat16

_PERM = tuple(32 * (j // 32) + (j % 32) // 2 + 16 * (j % 2)
              for j in range(64))
_PERM_FULL = _PERM + tuple(64 + p for p in _PERM)


def _sc_aggr_body(x_hbm, src_hbm, dst_hbm, ew_hbm, out_hbm,
                  src_v, dst_v, x_v, ew_v, m_v, zero_v, gsem, esem, ssem,
                  aggr_sh):
    cid = lax.axis_index("c")
    sid = lax.axis_index("s")

    pltpu.sync_copy(src_hbm.at[sid], src_v)
    pltpu.sync_copy(dst_hbm.at[sid], dst_v)

    def _zrow(r, c):
        for cc in range(_HD // 16):
            zero_v[r, pl.ds(cc * 16, 16)] = jnp.zeros((16,), _F32)
        return c
    lax.fori_loop(0, _ZROWS, _zrow, 0)
    zbase = pl.multiple_of(sid * _STRIDE, 8)
    for j in range(_SWIDTH // _ZROWS):
        pltpu.sync_copy(zero_v, aggr_sh.at[pl.ds(zbase + j * _ZROWS, _ZROWS)])
    plsc.subcore_barrier()

    def _fire(ch, b):
        pltpu.async_copy(x_hbm.at[cid].at[src_v.at[ch]], x_v.at[b], gsem.at[b])
        ebase = sid * (_EPT // 2) + ch * (_CHUNK // 2)
        pltpu.async_copy(ew_hbm.at[cid, pl.ds(ebase, _CHUNK // 2)],
                         ew_v.at[b], esem.at[b])

    def _wait(b):
        pltpu.make_async_copy(x_hbm.at[cid, pl.ds(0, _CHUNK)], x_v.at[b],
                              gsem.at[b]).wait()
        pltpu.make_async_copy(ew_hbm.at[cid, pl.ds(0, _CHUNK // 2)],
                              ew_v.at[b], esem.at[b]).wait()

    def _swait(ch, b):
        pltpu.make_async_copy(m_v.at[b], aggr_sh.at[dst_v.at[ch]],
                              ssem.at[b]).wait()

    def _process(i, ch, b):
        _wait(b)

        @pl.when(i > 0)
        def _():
            _swait(ch, b)

        def _rows(rq, c2):
            for rr in range(5):
                r = rq * 5 + rr
                for h in range(2):
                    e = r * 2 + h
                    for cc in range(_HD // 16):
                        s = pl.ds(cc * 16, 16)
                        se = pl.ds(h * _HD + cc * 16, 16)
                        m_v[b, e, s] = jnp.maximum(x_v[b, e, s] +
                                                   ew_v[b, r, se], 0.0)
            return c2
        lax.fori_loop(0, _CHUNK // 10, _rows, 0)
        pltpu.async_copy(m_v.at[b], aggr_sh.at[dst_v.at[ch]], ssem.at[b],
                         add=True)

    _fire(0, 0)
    npair = _NCH // 2

    def _pair(i, c):
        c0 = 2 * i
        _fire(c0 + 1, 1)
        _process(i, c0, 0)

        @pl.when(i < npair - 1)
        def _():
            _fire(c0 + 2, 0)
        _process(i, c0 + 1, 1)
        return c
    lax.fori_loop(0, npair, _pair, 0)
    for b in range(2):
        _swait(0, b)
    plsc.subcore_barrier()

    obase = pl.multiple_of(sid * _STRIDE, 8)
    pltpu.sync_copy(aggr_sh.at[pl.ds(obase, _SWIDTH)],
                    out_hbm.at[cid, pl.ds(obase, _SWIDTH)])


def _sc_aggregate(xs, src3, dst3, ews):
    mesh = plsc.VectorSubcoreMesh(core_axis_name="c", subcore_axis_name="s")
    fn = functools.partial(
        pl.kernel,
        out_type=jax.ShapeDtypeStruct((_SC_CORES, _N, _HD), _F32),
        mesh=mesh,
        scratch_types=[
            pltpu.VMEM((_NCH, _CHUNK), jnp.int32),
            pltpu.VMEM((_NCH, _CHUNK), jnp.int32),
            pltpu.VMEM((2, _CHUNK, _HD), _F32),
            pltpu.VMEM((2, _CHUNK // 2, _D), _F32),
            pltpu.VMEM((2, _CHUNK, _HD), _F32),
            pltpu.VMEM((_ZROWS, _HD), _F32),
            pltpu.SemaphoreType.DMA((2,)),
            pltpu.SemaphoreType.DMA((2,)),
            pltpu.SemaphoreType.DMA((2,)),
            pltpu.VMEM_SHARED((_N, _HD), _F32),
        ],
        compiler_params=pltpu.CompilerParams(use_tc_tiling_on_sc=False),
    )(_sc_aggr_body)
    return fn(xs, src3, dst3, ews)


_ET_BLK = 8000


_EP_BLK = _ET_BLK // 2


def _edge_tf_body(eap_ref, w00_ref, w01_ref, w10_ref, w11_ref,
                  b00_ref, b01_ref, b10_ref, b11_ref, e0_ref, e1_ref):
    eap = eap_ref[...]
    e00 = jnp.dot(eap, w00_ref[...], preferred_element_type=_F32) + b00_ref[...]
    e01 = jnp.dot(eap, w01_ref[...], preferred_element_type=_F32) + b01_ref[...]
    e10 = jnp.dot(eap, w10_ref[...], preferred_element_type=_F32) + b10_ref[...]
    e11 = jnp.dot(eap, w11_ref[...], preferred_element_type=_F32) + b11_ref[...]
    e0_ref[...] = jnp.stack([e00, e01])
    e1_ref[...] = jnp.stack([e10, e11])


_EP_GRID = (_E // 2) // _EP_BLK


def _edge_transform(eap, wps, bps):
    return pl.pallas_call(
        _edge_tf_body,
        grid=(_EP_GRID,),
        in_specs=[
            pl.BlockSpec((_EP_BLK, 2 * _ED), lambda i: (i, 0)),
            pl.BlockSpec((2 * _ED, _D), lambda i: (0, 0)),
            pl.BlockSpec((2 * _ED, _D), lambda i: (0, 0)),
            pl.BlockSpec((2 * _ED, _D), lambda i: (0, 0)),
            pl.BlockSpec((2 * _ED, _D), lambda i: (0, 0)),
            pl.BlockSpec((1, _D), lambda i: (0, 0)),
            pl.BlockSpec((1, _D), lambda i: (0, 0)),
            pl.BlockSpec((1, _D), lambda i: (0, 0)),
            pl.BlockSpec((1, _D), lambda i: (0, 0)),
        ],
        out_specs=[
            pl.BlockSpec((2, _EP_BLK, _D), lambda i: (0, i, 0)),
            pl.BlockSpec((2, _EP_BLK, _D), lambda i: (0, i, 0)),
        ],
        out_shape=[
            jax.ShapeDtypeStruct((2, _E // 2, _D), _F32),
            jax.ShapeDtypeStruct((2, _E // 2, _D), _F32),
        ],
    )(eap, *wps, *bps)


_NU_BLK = 2000


def _node_update_body(x_ref, p_ref, w1_ref, b1_ref, w2_ref, b2_ref,
                      u_ref, st_ref):
    i = pl.program_id(0)
    p = p_ref[...]
    h = x_ref[...] + jnp.concatenate([p[0], p[1]], axis=1)
    t = jnp.maximum(jnp.dot(h, w1_ref[...], preferred_element_type=_F32)
                    + b1_ref[...], 0.0)
    u = jnp.dot(t, w2_ref[...], preferred_element_type=_F32) + b2_ref[...]
    u_ref[...] = u
    s = jnp.concatenate([jnp.sum(u, axis=0, keepdims=True),
                         jnp.sum(u * u, axis=0, keepdims=True)], axis=0)

    @pl.when(i == 0)
    def _():
        st_ref[...] = s

    @pl.when(i > 0)
    def _():
        st_ref[...] = st_ref[...] + s


def _node_update(x, parts, w1t, b1, w2t, b2):
    g = _N // _NU_BLK
    return pl.pallas_call(
        _node_update_body,
        grid=(g,),
        in_specs=[
            pl.BlockSpec((_NU_BLK, _D), lambda i: (i, 0)),
            pl.BlockSpec((2, _NU_BLK, _HD), lambda i: (0, i, 0)),
            pl.BlockSpec((_D, _H), lambda i: (0, 0)),
            pl.BlockSpec((1, _H), lambda i: (0, 0)),
            pl.BlockSpec((_H, _H), lambda i: (0, 0)),
            pl.BlockSpec((1, _H), lambda i: (0, 0)),
        ],
        out_specs=[
            pl.BlockSpec((_NU_BLK, _H), lambda i: (i, 0)),
            pl.BlockSpec((2, _H), lambda i: (0, 0)),
        ],
        out_shape=[
            jax.ShapeDtypeStruct((_N, _H), _F32),
            jax.ShapeDtypeStruct((2, _H), _F32),
        ],
    )(x, parts, w1t, b1, w2t, b2)


def _bn_relu_body(u_ref, st_ref, g_ref, b_ref, y_ref, ys_ref):
    st = st_ref[...]
    mu = st[0:1] / _N
    var = st[1:2] / _N - mu * mu
    inv = lax.rsqrt(var + 1e-5) * g_ref[...]
    y = jnp.maximum((u_ref[...] - mu) * inv + b_ref[...], 0.0)
    y_ref[...] = y
    ys_ref[...] = jnp.stack([y[:, :_HD], y[:, _HD:]])


def _bn_relu(u, st, g, b):
    gr = _N // _NU_BLK
    return pl.pallas_call(
        _bn_relu_body,
        grid=(gr,),
        in_specs=[
            pl.BlockSpec((_NU_BLK, _H), lambda i: (i, 0)),
            pl.BlockSpec((2, _H), lambda i: (0, 0)),
            pl.BlockSpec((1, _H), lambda i: (0, 0)),
            pl.BlockSpec((1, _H), lambda i: (0, 0)),
        ],
        out_specs=[
            pl.BlockSpec((_NU_BLK, _H), lambda i: (i, 0)),
            pl.BlockSpec((2, _NU_BLK, _HD), lambda i: (0, i, 0)),
        ],
        out_shape=[
            jax.ShapeDtypeStruct((_N, _H), _F32),
            jax.ShapeDtypeStruct((2, _N, _HD), _F32),
        ],
    )(u, st, g, b)


_PB = _NU_BLK
_PG = _N // _PB


def _pool_body(u_ref, st_ref, g_ref, b_ref, batch_ref, emb_ref,
               acc_ref, cnt_ref):
    i = pl.program_id(0)

    @pl.when(i == 0)
    def _():
        acc_ref[...] = jnp.zeros_like(acc_ref)
        cnt_ref[...] = jnp.zeros_like(cnt_ref)

    st = st_ref[...]
    mu = st[0:1] / _N
    var = st[1:2] / _N - mu * mu
    inv = lax.rsqrt(var + 1e-5) * g_ref[...]
    y = jnp.maximum((u_ref[...] - mu) * inv + b_ref[...], 0.0)
    bb = batch_ref[0]
    oht = (bb == lax.broadcasted_iota(jnp.int32, (_B, _PB), 0)
           ).astype(_F32)
    acc_ref[...] += jnp.dot(oht, y, preferred_element_type=_F32)
    cnt_ref[...] += jnp.sum(oht, axis=1, keepdims=True)

    @pl.when(i == _PG - 1)
    def _():
        emb_ref[...] = acc_ref[...] / jnp.maximum(cnt_ref[...], 1.0)


def _pool(u, st, g, b, batch3):
    return pl.pallas_call(
        _pool_body,
        grid=(_PG,),
        in_specs=[
            pl.BlockSpec((_PB, _H), lambda i: (i, 0)),
            pl.BlockSpec((2, _H), lambda i: (0, 0)),
            pl.BlockSpec((1, _H), lambda i: (0, 0)),
            pl.BlockSpec((1, _H), lambda i: (0, 0)),
            pl.BlockSpec((1, 1, _PB), lambda i: (i, 0, 0)),
        ],
        out_specs=pl.BlockSpec((_B, _H), lambda i: (0, 0)),
        out_shape=jax.ShapeDtypeStruct((_B, _H), _F32),
        scratch_shapes=[
            pltpu.VMEM((_B, _H), _F32),
            pltpu.VMEM((_B, 1), _F32),
        ],
    )(u, st, g, b, batch3)


def _head_body(e1_ref, e2_ref, cv_ref, ci_ref, tab_ref,
               w1_ref, b1_ref, w2_ref, b2_ref, w3_ref, b3_ref, o_ref):
    ci = ci_ref[...]
    oh = (ci == lax.broadcasted_iota(jnp.int32, (_B, _NCELL), 1)
          ).astype(_F32)
    ce = jnp.dot(oh, tab_ref[...], preferred_element_type=_F32)
    comb = jnp.concatenate([e1_ref[...], e2_ref[...], cv_ref[...], ce],
                           axis=1)
    h = jnp.maximum(jnp.dot(comb, w1_ref[...], preferred_element_type=_F32)
                    + b1_ref[...], 0.0)
    h = jnp.maximum(jnp.dot(h, w2_ref[...], preferred_element_type=_F32)
                    + b2_ref[...], 0.0)
    o_ref[...] = jnp.dot(h, w3_ref[...], preferred_element_type=_F32) + b3_ref[...]


def _head(e1, e2, cv, ci_col, tab, w1t, b1, w2t, b2, w3t, b3):
    return pl.pallas_call(
        _head_body,
        out_shape=jax.ShapeDtypeStruct((_B, 1), _F32),
    )(e1, e2, cv, ci_col, tab, w1t, b1, w2t, b2, w3t, b3)


def _encode(x, xs, src3, dst3, ea, batch3, lp):
    ew0, ew1 = _edge_transform(ea, lp["wps"], lp["bps"])
    p0 = _sc_aggregate(xs, src3, dst3, ew0)
    u0, st0 = _node_update(x, p0, lp["w01t"], lp["b01"], lp["w02t"], lp["b02"])
    y0, y0s = _bn_relu(u0, st0, lp["g0"], lp["bb0"])
    p1 = _sc_aggregate(y0s, src3, dst3, ew1)
    u1, st1 = _node_update(y0, p1, lp["w11t"], lp["b11"], lp["w12t"], lp["b12"])
    return _pool(u1, st1, lp["g1"], lp["bb1"], batch3)


def kernel(x1, edge_index1, edge_attr1, batch1, x2, edge_index2, edge_attr2,
           batch2, cancer_vec, cell_idx,
           conv0_We, conv0_be, conv0_W1, conv0_b1, conv0_W2, conv0_b2,
           bn0_g, bn0_b,
           conv1_We, conv1_be, conv1_W1, conv1_b1, conv1_W2, conv1_b2,
           bn1_g, bn1_b,
           cell_table, fc1_W, fc1_b, fc2_W, fc2_b, fc3_W, fc3_b):
    def _pw(wt, c):
        h = wt[:, c * _HD:(c + 1) * _HD]
        z = jnp.zeros((2 * _ED, _D), _F32)
        return z.at[:_ED, :_HD].set(h).at[_ED:, _HD:].set(h)

    def _pb(b, c):
        h = b.reshape(1, -1)[:, c * _HD:(c + 1) * _HD]
        return jnp.concatenate([h, h], axis=1)

    w0et = conv0_We.T
    w1et = conv1_We.T
    lp = {
        "wps": [_pw(w0et, 0), _pw(w0et, 1), _pw(w1et, 0), _pw(w1et, 1)],
        "bps": [_pb(conv0_be, 0), _pb(conv0_be, 1),
                _pb(conv1_be, 0), _pb(conv1_be, 1)],
        "w01t": conv0_W1.T, "b01": conv0_b1.reshape(1, _H),
        "w02t": conv0_W2.T, "b02": conv0_b2.reshape(1, _H),
        "w11t": conv1_W1.T, "b11": conv1_b1.reshape(1, _H),
        "w12t": conv1_W2.T, "b12": conv1_b2.reshape(1, _H),
        "g0": bn0_g.reshape(1, _H), "bb0": bn0_b.reshape(1, _H),
        "g1": bn1_g.reshape(1, _H), "bb1": bn1_b.reshape(1, _H),
    }
    src3a = edge_index1[0].reshape(_SC_TILES, _NCH, _CHUNK)
    dst3a = edge_index1[1].reshape(_SC_TILES, _NCH, _CHUNK)
    src3b = edge_index2[0].reshape(_SC_TILES, _NCH, _CHUNK)
    dst3b = edge_index2[1].reshape(_SC_TILES, _NCH, _CHUNK)
    batch3a = batch1.reshape(_PG, 1, _PB)
    batch3b = batch2.reshape(_PG, 1, _PB)
    x1s = jnp.stack([x1[:, :_HD], x1[:, _HD:]])
    x2s = jnp.stack([x2[:, :_HD], x2[:, _HD:]])
    eap1 = edge_attr1.reshape(_E // 2, 2 * _ED)
    eap2 = edge_attr2.reshape(_E // 2, 2 * _ED)

    emb1 = _encode(x1, x1s, src3a, dst3a, eap1, batch3a, lp)
    emb2 = _encode(x2, x2s, src3b, dst3b, eap2, batch3b, lp)

    return _head(emb1, emb2, cancer_vec, cell_idx.reshape(_B, 1), cell_table,
                 fc1_W.T, fc1_b.reshape(1, _MH),
                 fc2_W.T, fc2_b.reshape(1, _MH // 2),
                 fc3_W.T, fc3_b.reshape(1, 1))
